# baseline (speedup 1.0000x reference)
import jax
import jax.numpy as jnp
from jax.experimental import pallas as pl
from jax.experimental.pallas import tpu as pltpu

_BATCH = 4


_SPLIT = 4


def _copy_kernel(tbl, out, buf, in_sem, out_sem):
    num_rows = buf.shape[0]
    chunk = num_rows // _SPLIT
    pltpu.make_async_copy(tbl, buf, in_sem).start()
    pltpu.make_async_copy(tbl, buf, in_sem).wait()
    copies = []
    for b in range(_BATCH):
        for c in range(_SPLIT):
            sl = pl.ds(c * chunk, chunk)
            copies.append(pltpu.make_async_copy(
                buf.at[sl, :], out.at[b, sl, :], out_sem.at[b, c]))
    for cp in copies:
        cp.start()
    for cp in copies:
        cp.wait()


def kernel(inputs, table):
    del inputs
    num_rows, dim = table.shape
    out = pl.pallas_call(
        _copy_kernel,
        in_specs=[pl.BlockSpec(memory_space=pl.ANY)],
        out_specs=pl.BlockSpec(memory_space=pl.ANY),
        out_shape=jax.ShapeDtypeStruct((_BATCH, num_rows, dim), table.dtype),
        scratch_shapes=[
            pltpu.VMEM((num_rows, dim), table.dtype),
            pltpu.SemaphoreType.DMA,
            pltpu.SemaphoreType.DMA((_BATCH, _SPLIT)),
        ],
    )(table)
    return out


# speedup vs baseline: 1.0582x; 1.0582x over previous
import jax
import jax.numpy as jnp
from jax.experimental import pallas as pl
from jax.experimental.pallas import tpu as pltpu

_BATCH = 4
_BLK = 4096


def _copy_kernel(tbl, out, buf, in_sem, out_sem):
    n = pl.num_programs(0)
    j = pl.program_id(0)
    slot = j % 2
    nslot = (j + 1) % 2

    def in_copy(blk_idx, s):
        return pltpu.make_async_copy(
            tbl.at[pl.ds(blk_idx * _BLK, _BLK), :], buf.at[s], in_sem.at[s])

    def out_copy(b, blk_idx, s):
        return pltpu.make_async_copy(
            buf.at[s], out.at[b, pl.ds(blk_idx * _BLK, _BLK), :],
            out_sem.at[s, b])

    @pl.when(j == 0)
    def _():
        in_copy(0, 0).start()

    @pl.when(j + 1 < n)
    def _():
        @pl.when(j >= 1)
        def _():
            for b in range(_BATCH):
                out_copy(b, j - 1, nslot).wait()

        in_copy(j + 1, nslot).start()

    in_copy(j, slot).wait()
    for b in range(_BATCH):
        out_copy(b, j, slot).start()

    @pl.when(j == n - 1)
    def _():
        for b in range(_BATCH):
            out_copy(b, j, slot).wait()

        @pl.when(n > 1)
        def _():
            for b in range(_BATCH):
                out_copy(b, j - 1, nslot).wait()


def kernel(inputs, table):
    del inputs
    num_rows, dim = table.shape
    grid = (num_rows // _BLK,)
    out = pl.pallas_call(
        _copy_kernel,
        grid=grid,
        in_specs=[pl.BlockSpec(memory_space=pl.ANY)],
        out_specs=pl.BlockSpec(memory_space=pl.ANY),
        out_shape=jax.ShapeDtypeStruct((_BATCH, num_rows, dim), table.dtype),
        scratch_shapes=[
            pltpu.VMEM((2, _BLK, dim), table.dtype),
            pltpu.SemaphoreType.DMA((2,)),
            pltpu.SemaphoreType.DMA((2, _BATCH)),
        ],
    )(table)
    return out
